# baseline (speedup 1.0000x reference)
import numpy as np
import jax
import jax.numpy as jnp
from jax.experimental import pallas as pl
from jax.experimental.pallas import tpu as pltpu

_GAMMA = 0.9865
_DELTA = _GAMMA * _GAMMA
_CHUNK = 256


def _retention_body(xq_ref, xk_ref, xv_ref, wq_ref, wk_ref, wv_ref,
                    dmat_ref, qdec_ref, kdec_ref, out_ref, r_ref, stage_ref):
    c = pl.program_id(1)
    n = pl.num_programs(1) - 1
    C = _CHUNK
    f32 = jnp.float32

    @pl.when(c == 0)
    def _():
        r_ref[...] = jnp.zeros_like(r_ref)

    @pl.when(c < n)
    def _compute():
        q = jnp.dot(xq_ref[0], wq_ref[...], preferred_element_type=f32)
        k = jnp.dot(xk_ref[0], wk_ref[...], preferred_element_type=f32)
        v = jnp.dot(xv_ref[0], wv_ref[...], preferred_element_type=f32)

        ones = jnp.ones((q.shape[1], 128), f32)
        qsum = jnp.dot(q, ones, preferred_element_type=f32)
        ksum = jnp.dot(k, ones, preferred_element_type=f32)
        qk = qsum * ksum
        qkd = jnp.concatenate([qk, qk, qk, qk], axis=1)
        diag = qkd * v

        a = jax.lax.dot_general(q, k, (((1,), (1,)), ((), ())),
                                preferred_element_type=f32)
        a = a * dmat_ref[...]
        ysh = jnp.dot(a, v, preferred_element_type=f32) + jnp.dot(
            q * qdec_ref[...], r_ref[...], preferred_element_type=f32)

        @pl.when(c > 0)
        def _():
            stage_ref[C - 1:C, :] = stage_ref[C - 1:C, :] + ysh[0:1, :]
        out_ref[0] = stage_ref[...]

        stage_ref[0:C - 1, :] = diag[0:C - 1, :] + ysh[1:C, :]
        stage_ref[C - 1:C, :] = diag[C - 1:C, :]

        r_ref[...] = f32(_DELTA ** C) * r_ref[...] + jax.lax.dot_general(
            k * kdec_ref[...], v, (((0,), (0,)), ((), ())),
            preferred_element_type=f32)

    @pl.when(c == n)
    def _flush():
        out_ref[0] = stage_ref[...]


@jax.jit
def kernel(xq, xk, xv, Wq, Wk, Wv):
    B, S, D = xq.shape
    C = _CHUNK
    N = S // C

    i = np.arange(C)
    dmat = np.where(i[:, None] > i[None, :],
                    _DELTA ** (i[:, None] - i[None, :] + 0.0),
                    0.0).astype(np.float32)
    qdec = (_DELTA ** (i + 1.0)).astype(np.float32).reshape(C, 1)
    kdec = (_DELTA ** (C - 1.0 - i)).astype(np.float32).reshape(C, 1)

    def in_map(b, c):
        return (b, jnp.minimum(c, N - 1), 0)

    def w_map(b, c):
        return (0, 0)

    def out_map(b, c):
        return (b, jnp.maximum(c - 1, 0), 0)

    return pl.pallas_call(
        _retention_body,
        grid=(B, N + 1),
        in_specs=[
            pl.BlockSpec((1, C, D), in_map),
            pl.BlockSpec((1, C, D), in_map),
            pl.BlockSpec((1, C, D), in_map),
            pl.BlockSpec((D, D), w_map),
            pl.BlockSpec((D, D), w_map),
            pl.BlockSpec((D, D), w_map),
            pl.BlockSpec((C, C), w_map),
            pl.BlockSpec((C, 1), w_map),
            pl.BlockSpec((C, 1), w_map),
        ],
        out_specs=pl.BlockSpec((1, C, D), out_map),
        out_shape=jax.ShapeDtypeStruct((B, S, D), jnp.float32),
        scratch_shapes=[pltpu.VMEM((D, D), jnp.float32),
                        pltpu.VMEM((C, D), jnp.float32)],
        compiler_params=pltpu.CompilerParams(
            dimension_semantics=("parallel", "arbitrary"),
            vmem_limit_bytes=96 * 1024 * 1024,
        ),
        name="chunkwise_retention",
    )(xq, xk, xv, Wq, Wk, Wv,
      jnp.asarray(dmat), jnp.asarray(qdec), jnp.asarray(kdec))
